# baseline (speedup 1.0000x reference)
import jax
import jax.numpy as jnp
from jax.experimental import pallas as pl
from jax.experimental.pallas import tpu as pltpu


def _round_up(v, m):
    return ((v + m - 1) // m) * m


def _prep_kernel(a_ref, x_ref, w1_ref, ab_ref, ns_ref, p1_ref):
    a = a_ref[...]
    ab_ref[...] = a.astype(jnp.float8_e4m3fn)
    deg = jnp.sum(a, axis=1, keepdims=True)
    ns = jnp.where(deg > 0, jax.lax.rsqrt(deg), 0.0)
    ns_ref[...] = ns
    p = jnp.dot(x_ref[...].astype(jnp.bfloat16), w1_ref[...].astype(jnp.bfloat16),
                preferred_element_type=jnp.float32)
    p1_ref[...] = (p * ns).astype(jnp.bfloat16)


def _prep(a, x, w1, *, block_m):
    n_pad = a.shape[0]
    f_pad = x.shape[1]
    h_pad = w1.shape[1]
    est = (2 * block_m * n_pad * 4
           + 2 * block_m * n_pad * 1
           + 2 * block_m * f_pad * 4 + f_pad * h_pad * 2
           + 2 * block_m * h_pad * 2 + (2 << 20))
    return pl.pallas_call(
        _prep_kernel,
        out_shape=(
            jax.ShapeDtypeStruct((n_pad, n_pad), jnp.float8_e4m3fn),
            jax.ShapeDtypeStruct((n_pad, 1), jnp.float32),
            jax.ShapeDtypeStruct((n_pad, h_pad), jnp.bfloat16),
        ),
        grid=(n_pad // block_m,),
        in_specs=[
            pl.BlockSpec((block_m, n_pad), lambda i: (i, 0)),
            pl.BlockSpec((block_m, f_pad), lambda i: (i, 0)),
            pl.BlockSpec((f_pad, h_pad), lambda i: (0, 0)),
        ],
        out_specs=(
            pl.BlockSpec((block_m, n_pad), lambda i: (i, 0)),
            pl.BlockSpec((block_m, 1), lambda i: (i, 0)),
            pl.BlockSpec((block_m, h_pad), lambda i: (i, 0)),
        ),
        compiler_params=pltpu.CompilerParams(
            dimension_semantics=("parallel",),
            vmem_limit_bytes=min(est, 100 << 20),
        ),
    )(a, x, w1)


def _layer1_kernel(ab_ref, p1_ref, ns_ref, b1_ref, w2_ref, p2_ref):
    acc = jnp.dot(ab_ref[...].astype(jnp.bfloat16), p1_ref[...],
                  preferred_element_type=jnp.float32)
    ns = ns_ref[...]
    h = jnp.maximum(acc * ns + b1_ref[...], 0.0)
    p2 = jnp.dot(h.astype(jnp.bfloat16), w2_ref[...].astype(jnp.bfloat16),
                 preferred_element_type=jnp.float32)
    p2_ref[...] = (p2 * ns).astype(jnp.bfloat16)


def _layer1(ab, p1, ns, b1, w2, *, block_m):
    n_pad = ab.shape[0]
    h_pad = p1.shape[1]
    est = (2 * block_m * n_pad * 1 + 2 * n_pad * h_pad * 2
           + h_pad * h_pad * 2 + 2 * block_m * h_pad * 2 + (2 << 20))
    return pl.pallas_call(
        _layer1_kernel,
        out_shape=jax.ShapeDtypeStruct((n_pad, h_pad), jnp.bfloat16),
        grid=(n_pad // block_m,),
        in_specs=[
            pl.BlockSpec((block_m, n_pad), lambda i: (i, 0)),
            pl.BlockSpec((n_pad, h_pad), lambda i: (0, 0)),
            pl.BlockSpec((block_m, 1), lambda i: (i, 0)),
            pl.BlockSpec((1, h_pad), lambda i: (0, 0)),
            pl.BlockSpec((h_pad, h_pad), lambda i: (0, 0)),
        ],
        out_specs=pl.BlockSpec((block_m, h_pad), lambda i: (i, 0)),
        compiler_params=pltpu.CompilerParams(
            dimension_semantics=("parallel",),
            vmem_limit_bytes=min(est, 100 << 20),
        ),
    )(ab, p1, ns, b1, w2)


def _layer2_kernel(ab_ref, p2_ref, ns_ref, b2_ref, out_ref):
    acc = jnp.dot(ab_ref[...].astype(jnp.bfloat16), p2_ref[...],
                  preferred_element_type=jnp.float32)
    out_ref[...] = acc * ns_ref[...] + b2_ref[...]


def _layer2(ab, p2, ns, b2, *, block_m):
    n_pad = ab.shape[0]
    h_pad = p2.shape[1]
    est = (2 * block_m * n_pad * 1 + 2 * n_pad * h_pad * 2
           + 2 * block_m * h_pad * 4 + (2 << 20))
    return pl.pallas_call(
        _layer2_kernel,
        out_shape=jax.ShapeDtypeStruct((n_pad, h_pad), jnp.float32),
        grid=(n_pad // block_m,),
        in_specs=[
            pl.BlockSpec((block_m, n_pad), lambda i: (i, 0)),
            pl.BlockSpec((n_pad, h_pad), lambda i: (0, 0)),
            pl.BlockSpec((block_m, 1), lambda i: (i, 0)),
            pl.BlockSpec((1, h_pad), lambda i: (0, 0)),
        ],
        out_specs=pl.BlockSpec((block_m, h_pad), lambda i: (i, 0)),
        compiler_params=pltpu.CompilerParams(
            dimension_semantics=("parallel",),
            vmem_limit_bytes=min(est, 100 << 20),
        ),
    )(ab, p2, ns, b2)


def kernel(a, x, w1, b1, w2, b2):
    n, f = x.shape
    h_feats = w1.shape[1]

    f_pad = _round_up(f, 128)
    h_pad = _round_up(h_feats, 128)
    block_m = min(1024, _round_up(n, 128))
    block_l = min(2048, _round_up(n, 128))
    n_pad = _round_up(n, block_l if block_l > block_m else block_m)

    def pad2(m, r, c):
        return jnp.pad(m, ((0, r - m.shape[0]), (0, c - m.shape[1])))

    a_p = pad2(a, n_pad, n_pad)
    x_p = pad2(x, n_pad, f_pad)
    w1_p = pad2(w1, f_pad, h_pad)
    w2_p = pad2(w2, h_pad, h_pad)
    b1_p = pad2(b1, 1, h_pad)
    b2_p = pad2(b2, 1, h_pad)

    ab, ns, p1 = _prep(a_p, x_p, w1_p, block_m=block_m)
    p2 = _layer1(ab, p1, ns, b1_p, w2_p, block_m=block_l)
    h2 = _layer2(ab, p2, ns, b2_p, block_m=block_l)
    return h2[:n, :h_feats]


# speedup vs baseline: 1.4648x; 1.4648x over previous
import jax
import jax.numpy as jnp
from jax.experimental import pallas as pl
from jax.experimental.pallas import tpu as pltpu


def _round_up(v, m):
    return ((v + m - 1) // m) * m


def _prep_kernel(a_ref, x_ref, w1_ref, ab_ref, ns_ref, p1_ref):
    a = a_ref[...]
    ab_ref[...] = a.astype(jnp.float8_e4m3fn)
    deg = jnp.sum(a, axis=1, keepdims=True)
    ns = jnp.where(deg > 0, jax.lax.rsqrt(deg), 0.0)
    ns_ref[...] = ns
    p = jnp.dot(x_ref[...].astype(jnp.bfloat16), w1_ref[...].astype(jnp.bfloat16),
                preferred_element_type=jnp.float32)
    p1_ref[...] = (p * ns).astype(jnp.bfloat16)


def _prep(a, x, w1, *, block_m):
    n_pad = a.shape[0]
    f_pad = x.shape[1]
    h_pad = w1.shape[1]
    est = (2 * block_m * n_pad * 4
           + 2 * block_m * n_pad * 1
           + 2 * block_m * f_pad * 4 + f_pad * h_pad * 2
           + 2 * block_m * h_pad * 2 + (2 << 20))
    return pl.pallas_call(
        _prep_kernel,
        out_shape=(
            jax.ShapeDtypeStruct((n_pad, n_pad), jnp.float8_e4m3fn),
            jax.ShapeDtypeStruct((n_pad, 1), jnp.float32),
            jax.ShapeDtypeStruct((n_pad, h_pad), jnp.bfloat16),
        ),
        grid=(n_pad // block_m,),
        in_specs=[
            pl.BlockSpec((block_m, n_pad), lambda i: (i, 0)),
            pl.BlockSpec((block_m, f_pad), lambda i: (i, 0)),
            pl.BlockSpec((f_pad, h_pad), lambda i: (0, 0)),
        ],
        out_specs=(
            pl.BlockSpec((block_m, n_pad), lambda i: (i, 0)),
            pl.BlockSpec((block_m, 1), lambda i: (i, 0)),
            pl.BlockSpec((block_m, h_pad), lambda i: (i, 0)),
        ),
        compiler_params=pltpu.CompilerParams(
            dimension_semantics=("parallel",),
            vmem_limit_bytes=min(est, 100 << 20),
        ),
    )(a, x, w1)


def _layer1_kernel(ab_ref, p1_ref, ns_ref, b1_ref, w2_ref, p2_ref):
    acc = jnp.dot(ab_ref[...].astype(jnp.bfloat16), p1_ref[...],
                  preferred_element_type=jnp.float32)
    ns = ns_ref[...]
    h = jnp.maximum(acc * ns + b1_ref[...], 0.0)
    p2 = jnp.dot(h.astype(jnp.bfloat16), w2_ref[...].astype(jnp.bfloat16),
                 preferred_element_type=jnp.float32)
    p2_ref[...] = (p2 * ns).astype(jnp.bfloat16)


def _layer1(ab, p1, ns, b1, w2, *, block_m):
    n_pad = ab.shape[0]
    h_pad = p1.shape[1]
    est = (2 * block_m * n_pad * 1 + 2 * n_pad * h_pad * 2
           + h_pad * h_pad * 2 + 2 * block_m * h_pad * 2 + (2 << 20))
    return pl.pallas_call(
        _layer1_kernel,
        out_shape=jax.ShapeDtypeStruct((n_pad, h_pad), jnp.bfloat16),
        grid=(n_pad // block_m,),
        in_specs=[
            pl.BlockSpec((block_m, n_pad), lambda i: (i, 0)),
            pl.BlockSpec((n_pad, h_pad), lambda i: (0, 0)),
            pl.BlockSpec((block_m, 1), lambda i: (i, 0)),
            pl.BlockSpec((1, h_pad), lambda i: (0, 0)),
            pl.BlockSpec((h_pad, h_pad), lambda i: (0, 0)),
        ],
        out_specs=pl.BlockSpec((block_m, h_pad), lambda i: (i, 0)),
        compiler_params=pltpu.CompilerParams(
            dimension_semantics=("parallel",),
            vmem_limit_bytes=min(est, 100 << 20),
        ),
    )(ab, p1, ns, b1, w2)


def _layer2_kernel(ab_ref, p2_ref, ns_ref, b2_ref, out_ref):
    acc = jnp.dot(ab_ref[...].astype(jnp.bfloat16), p2_ref[...],
                  preferred_element_type=jnp.float32)
    out_ref[...] = acc * ns_ref[...] + b2_ref[...]


def _layer2(ab, p2, ns, b2, *, block_m):
    n_pad = ab.shape[0]
    h_pad = p2.shape[1]
    est = (2 * block_m * n_pad * 1 + 2 * n_pad * h_pad * 2
           + 2 * block_m * h_pad * 4 + (2 << 20))
    return pl.pallas_call(
        _layer2_kernel,
        out_shape=jax.ShapeDtypeStruct((n_pad, h_pad), jnp.float32),
        grid=(n_pad // block_m,),
        in_specs=[
            pl.BlockSpec((block_m, n_pad), lambda i: (i, 0)),
            pl.BlockSpec((n_pad, h_pad), lambda i: (0, 0)),
            pl.BlockSpec((block_m, 1), lambda i: (i, 0)),
            pl.BlockSpec((1, h_pad), lambda i: (0, 0)),
        ],
        out_specs=pl.BlockSpec((block_m, h_pad), lambda i: (i, 0)),
        compiler_params=pltpu.CompilerParams(
            dimension_semantics=("parallel",),
            vmem_limit_bytes=min(est, 100 << 20),
        ),
    )(ab, p2, ns, b2)


def kernel(a, x, w1, b1, w2, b2):
    n, f = x.shape
    h_feats = w1.shape[1]

    f_pad = _round_up(f, 128)
    h_pad = _round_up(h_feats, 128)
    block_m = min(512, _round_up(n, 128))
    block_l = min(2048, _round_up(n, 128))
    n_pad = _round_up(n, block_l if block_l > block_m else block_m)

    def pad2(m, r, c):
        return jnp.pad(m, ((0, r - m.shape[0]), (0, c - m.shape[1])))

    a_p = pad2(a, n_pad, n_pad)
    x_p = pad2(x, n_pad, f_pad)
    w1_p = pad2(w1, f_pad, h_pad)
    w2_p = pad2(w2, h_pad, h_pad)
    b1_p = pad2(b1, 1, h_pad)
    b2_p = pad2(b2, 1, h_pad)

    ab, ns, p1 = _prep(a_p, x_p, w1_p, block_m=block_m)
    p2 = _layer1(ab, p1, ns, b1_p, w2_p, block_m=block_l)
    h2 = _layer2(ab, p2, ns, b2_p, block_m=block_l)
    return h2[:n, :h_feats]
